# Initial kernel scaffold; baseline (speedup 1.0000x reference)
import jax
import jax.numpy as jnp
from jax.experimental import pallas as pl


def kernel(xyz, W1, b1, g1, be1, W2, b2, g2, be2, W3, b3):
    raise NotImplementedError("write your pallas kernel here")


# speedup vs baseline: 3.9163x; 3.9163x over previous
import functools

import jax
import jax.numpy as jnp
from jax import lax
from jax.experimental import pallas as pl
from jax.experimental.pallas import tpu as pltpu
from jax.experimental.pallas import tpu_sc as plsc

_B, _N, _C, _K = 4, 4096, 3, 16
_DIM_OUT = 64
_NC, _NS = 2, 16
_NW = _NC * _NS
_RPW = (_B * _N) // _NW
_WPB = _N // _RPW
_NCHUNK = _N // 16
_NK = _N * _K
_TE = 8192
_M = _B * _N * _K


def _sc_knn_body(xyzT_hbm, xbT_hbm, sq_hbm, planes_hbm, idx_hbm,
                 xv, yv, zv, xbv, ybv, zbv, sqv,
                 opx, opy, opz, onx, ony, onz, od2, oidx):
    cid = lax.axis_index("c")
    sid = lax.axis_index("s")
    wid = sid * _NC + cid
    b = wid // _WPB
    row0 = (wid % _WPB) * _RPW

    pltpu.sync_copy(xyzT_hbm.at[b, 0], xv)
    pltpu.sync_copy(xyzT_hbm.at[b, 1], yv)
    pltpu.sync_copy(xyzT_hbm.at[b, 2], zv)
    pltpu.sync_copy(xbT_hbm.at[b, 0], xbv)
    pltpu.sync_copy(xbT_hbm.at[b, 1], ybv)
    pltpu.sync_copy(xbT_hbm.at[b, 2], zbv)
    pltpu.sync_copy(sq_hbm.at[b], sqv)

    iota16 = lax.iota(jnp.int32, 16)
    inf = jnp.float32(jnp.inf)

    def row_body(r, carry_unused):
        q = row0 + r
        qvec = jnp.full((16,), q, jnp.int32)
        qx = plsc.load_gather(xv, [qvec])
        qy = plsc.load_gather(yv, [qvec])
        qz = plsc.load_gather(zv, [qvec])
        qxb = plsc.load_gather(xbv, [qvec])
        qyb = plsc.load_gather(ybv, [qvec])
        qzb = plsc.load_gather(zbv, [qvec])
        sqi = plsc.load_gather(sqv, [qvec])

        def chunk_body(j, carry):
            T, Tv, kth = carry
            c0 = j * 16
            cx = xbv[pl.ds(c0, 16)]
            cy = ybv[pl.ds(c0, 16)]
            cz = zbv[pl.ds(c0, 16)]
            cs = sqv[pl.ds(c0, 16)]
            dot = ((qxb * cx) + (qyb * cy)) + (qzb * cz)
            score = (sqi + cs) - 2.0 * dot

            def ins(_):
                sk, si = plsc.sort_key_val(score, c0 + iota16,
                                           descending=True)
                take = sk < T
                lk = jnp.where(take, sk, T)
                lv = jnp.where(take, si, Tv)
                t2, tv2 = plsc.sort_key_val(lk, lv)
                return t2, tv2, jnp.max(t2)

            def noins(_):
                return T, Tv, kth

            return lax.cond(jnp.any(score < kth), ins, noins, None)

        t0 = jnp.full((16,), inf, jnp.float32)
        tv0 = jnp.zeros((16,), jnp.int32)
        T, Tv, _ = lax.fori_loop(0, _NCHUNK, chunk_body, (t0, tv0, inf))

        nx = plsc.load_gather(xv, [Tv])
        ny = plsc.load_gather(yv, [Tv])
        nz = plsc.load_gather(zv, [Tv])
        dx = qx - nx
        dy = qy - ny
        dz = qz - nz
        d2 = (dx * dx + dy * dy) + dz * dz

        opx[r] = qx
        opy[r] = qy
        opz[r] = qz
        onx[r] = nx
        ony[r] = ny
        onz[r] = nz
        od2[r] = d2
        oidx[r] = Tv
        return 0

    lax.fori_loop(0, _RPW, row_body, 0)

    pltpu.sync_copy(opx, planes_hbm.at[b, 0, pl.ds(row0, _RPW)])
    pltpu.sync_copy(opy, planes_hbm.at[b, 1, pl.ds(row0, _RPW)])
    pltpu.sync_copy(opz, planes_hbm.at[b, 2, pl.ds(row0, _RPW)])
    pltpu.sync_copy(onx, planes_hbm.at[b, 3, pl.ds(row0, _RPW)])
    pltpu.sync_copy(ony, planes_hbm.at[b, 4, pl.ds(row0, _RPW)])
    pltpu.sync_copy(onz, planes_hbm.at[b, 5, pl.ds(row0, _RPW)])
    pltpu.sync_copy(od2, planes_hbm.at[b, 6, pl.ds(row0, _RPW)])
    pltpu.sync_copy(oidx, idx_hbm.at[b, pl.ds(row0, _RPW)])


@functools.cache
def _get_sc_knn():
    return functools.partial(
        pl.kernel,
        out_type=(jax.ShapeDtypeStruct((_B, 7, _N, _K), jnp.float32),
                  jax.ShapeDtypeStruct((_B, _N, _K), jnp.int32)),
        mesh=plsc.VectorSubcoreMesh(core_axis_name="c", subcore_axis_name="s",
                                    num_cores=_NC, num_subcores=_NS),
        scratch_types=(
            [pltpu.VMEM((_N,), jnp.float32)] * 7
            + [pltpu.VMEM((_RPW, _K), jnp.float32)] * 7
            + [pltpu.VMEM((_RPW, _K), jnp.int32)]
        ),
        compiler_params=pltpu.CompilerParams(use_tc_tiling_on_sc=False,
                                             needs_layout_passes=False),
    )(_sc_knn_body)


def _y1(pl_ref, a1w, b1w, w1d, b1):
    p3 = pl_ref[0, 0:3, :]
    n3 = pl_ref[0, 3:6, :]
    dist = jnp.sqrt(pl_ref[0, 6:7, :])
    y = lax.dot_general(a1w, p3, (((1,), (0,)), ((), ())),
                        precision=lax.Precision.HIGHEST)
    y = y + lax.dot_general(b1w, n3, (((1,), (0,)), ((), ())),
                            precision=lax.Precision.HIGHEST)
    return y + w1d * dist + b1


def _lrelu(x):
    return jnp.where(x >= 0, x, 0.01 * x)


def _accum(s_ref, q_ref, y):
    @pl.when((pl.program_id(0) == 0) & (pl.program_id(1) == 0))
    def _():
        s_ref[...] = jnp.zeros_like(s_ref)
        q_ref[...] = jnp.zeros_like(q_ref)

    s_ref[...] += jnp.sum(y, axis=1, keepdims=True)
    q_ref[...] += jnp.sum(y * y, axis=1, keepdims=True)


def _stats1_body(a1w_ref, b1w_ref, w1d_ref, b1_ref, pl_ref, s_ref, q_ref):
    y1 = _y1(pl_ref, a1w_ref[...], b1w_ref[...], w1d_ref[...], b1_ref[...])
    _accum(s_ref, q_ref, y1)


def _stats2_body(a1w_ref, b1w_ref, w1d_ref, b1_ref, a1_ref, c1_ref,
                 w2_ref, b2_ref, pl_ref, s_ref, q_ref):
    y1 = _y1(pl_ref, a1w_ref[...], b1w_ref[...], w1d_ref[...], b1_ref[...])
    h1 = _lrelu(a1_ref[...] * y1 + c1_ref[...])
    y2 = lax.dot_general(w2_ref[...], h1, (((1,), (0,)), ((), ())),
                         precision=lax.Precision.HIGHEST) + b2_ref[...]
    _accum(s_ref, q_ref, y2)


def _final_body(a1w_ref, b1w_ref, w1d_ref, b1_ref, a1_ref, c1_ref,
                w2_ref, b2_ref, a2_ref, c2_ref, w3_ref, b3_ref,
                pl_ref, out_ref):
    y1 = _y1(pl_ref, a1w_ref[...], b1w_ref[...], w1d_ref[...], b1_ref[...])
    h1 = _lrelu(a1_ref[...] * y1 + c1_ref[...])
    y2 = lax.dot_general(w2_ref[...], h1, (((1,), (0,)), ((), ())),
                         precision=lax.Precision.HIGHEST) + b2_ref[...]
    h2 = _lrelu(a2_ref[...] * y2 + c2_ref[...])
    f = lax.dot_general(w3_ref[...], h2, (((1,), (0,)), ((), ())),
                        precision=lax.Precision.HIGHEST) + b3_ref[...]
    out_ref[0] = f


def _small(shape):
    return pl.BlockSpec(shape, lambda b, e: (0,) * len(shape))


_PLANES_SPEC = pl.BlockSpec((1, 7, _TE), lambda b, e: (b, 0, e))
_GRID = (_B, _NK // _TE)
_STAT_OUT = [jax.ShapeDtypeStruct((64, 1), jnp.float32)] * 2
_STAT_OUT_SPEC = [pl.BlockSpec((64, 1), lambda b, e: (0, 0))] * 2
_W64 = _small((64, 3))
_V64 = _small((64, 1))


@functools.cache
def _get_tc_kernels():
    stats1 = pl.pallas_call(
        _stats1_body,
        grid=_GRID,
        in_specs=[_W64, _W64, _V64, _V64, _PLANES_SPEC],
        out_specs=_STAT_OUT_SPEC,
        out_shape=_STAT_OUT,
    )
    stats2 = pl.pallas_call(
        _stats2_body,
        grid=_GRID,
        in_specs=[_W64, _W64, _V64, _V64, _V64, _V64, _small((64, 64)), _V64,
                  _PLANES_SPEC],
        out_specs=_STAT_OUT_SPEC,
        out_shape=_STAT_OUT,
    )
    final = pl.pallas_call(
        _final_body,
        grid=_GRID,
        in_specs=[_W64, _W64, _V64, _V64, _V64, _V64, _small((64, 64)), _V64,
                  _V64, _V64, _small((64, 64)), _V64, _PLANES_SPEC],
        out_specs=pl.BlockSpec((1, 64, _TE), lambda b, e: (b, 0, e)),
        out_shape=jax.ShapeDtypeStruct((_B, 64, _NK), jnp.float32),
    )
    return stats1, stats2, final


def _bn_fold(s, q, g, be):
    m = s / _M
    v = q / _M - m * m
    a = g / jnp.sqrt(v + 1e-5)
    c = be - m * a
    return a, c


def kernel(xyz, W1, b1, g1, be1, W2, b2, g2, be2, W3, b3):
    xyzT = jnp.transpose(xyz, (0, 2, 1))
    sq = jnp.sum(xyz * xyz, axis=-1)
    xb = xyz.astype(jnp.bfloat16).astype(jnp.float32)
    xbT = jnp.transpose(xb, (0, 2, 1))
    planes, idxp = _get_sc_knn()(xyzT, xbT, sq)
    planes = planes.reshape(_B, 7, _NK)
    _stats1, _stats2, _final = _get_tc_kernels()

    a1w = W1[:, 0:3] + W1[:, 6:9]
    b1w = W1[:, 3:6] - W1[:, 6:9]
    w1d = W1[:, 9:10]
    b1r = b1.reshape(64, 1)
    b2r = b2.reshape(64, 1)
    b3r = b3.reshape(64, 1)

    s1, q1 = _stats1(a1w, b1w, w1d, b1r, planes)
    a1, c1 = _bn_fold(s1, q1, g1.reshape(64, 1), be1.reshape(64, 1))
    s2, q2 = _stats2(a1w, b1w, w1d, b1r, a1, c1, W2, b2r, planes)
    a2, c2 = _bn_fold(s2, q2, g2.reshape(64, 1), be2.reshape(64, 1))
    f = _final(a1w, b1w, w1d, b1r, a1, c1, W2, b2r, a2, c2, W3, b3r, planes)

    return (f.reshape(_B, _DIM_OUT, _N, _K), idxp.reshape(_B, _NK))


# speedup vs baseline: 8.0890x; 2.0655x over previous
import functools

import jax
import jax.numpy as jnp
from jax import lax
from jax.experimental import pallas as pl
from jax.experimental.pallas import tpu as pltpu
from jax.experimental.pallas import tpu_sc as plsc

_B, _N, _C, _K = 4, 4096, 3, 16
_DIM_OUT = 64
_NC, _NS = 2, 16
_NW = _NC * _NS
_RPW = (_B * _N) // _NW
_WPB = _N // _RPW
_NCHUNK = _N // 16
_NK = _N * _K
_TE = 8192
_M = _B * _N * _K


def _sc_knn_body(xyzT_hbm, xbT_hbm, sq_hbm, planes_hbm, idx_hbm,
                 xv, yv, zv, xbv, ybv, zbv, sqv, sbuf, cbs, cbi,
                 opx, opy, opz, onx, ony, onz, od2, oidx):
    cid = lax.axis_index("c")
    sid = lax.axis_index("s")
    wid = sid * _NC + cid
    b = wid // _WPB
    row0 = (wid % _WPB) * _RPW

    pltpu.sync_copy(xyzT_hbm.at[b, 0], xv)
    pltpu.sync_copy(xyzT_hbm.at[b, 1], yv)
    pltpu.sync_copy(xyzT_hbm.at[b, 2], zv)
    pltpu.sync_copy(xbT_hbm.at[b, 0], xbv)
    pltpu.sync_copy(xbT_hbm.at[b, 1], ybv)
    pltpu.sync_copy(xbT_hbm.at[b, 2], zbv)
    pltpu.sync_copy(sq_hbm.at[b], sqv)

    iota16 = lax.iota(jnp.int32, 16)
    inf = jnp.float32(jnp.inf)

    def row_body(r, carry_unused):
        q = row0 + r
        qvec = jnp.full((16,), q, jnp.int32)
        qx = plsc.load_gather(xv, [qvec])
        qy = plsc.load_gather(yv, [qvec])
        qz = plsc.load_gather(zv, [qvec])
        qxb = plsc.load_gather(xbv, [qvec])
        qyb = plsc.load_gather(ybv, [qvec])
        qzb = plsc.load_gather(zbv, [qvec])
        sqi = plsc.load_gather(sqv, [qvec])

        def p1_body(j, m):
            c0 = j * 16
            cx = xbv[pl.ds(c0, 16)]
            cy = ybv[pl.ds(c0, 16)]
            cz = zbv[pl.ds(c0, 16)]
            cs = sqv[pl.ds(c0, 16)]
            dot = ((qxb * cx) + (qyb * cy)) + (qzb * cz)
            score = (sqi + cs) - 2.0 * dot
            sbuf[pl.ds(c0, 16)] = score
            return jnp.minimum(m, score)

        m = lax.fori_loop(0, _NCHUNK, p1_body,
                          jnp.full((16,), inf, jnp.float32), unroll=8)
        tau = jnp.max(m)

        def p2_body(j, off):
            c0 = j * 16
            sc = sbuf[pl.ds(c0, 16)]
            msk = sc <= tau
            plsc.store_compressed(cbs.at[pl.ds(off, 16)], sc, mask=msk)
            plsc.store_compressed(cbi.at[pl.ds(off, 16)], c0 + iota16,
                                  mask=msk)
            return off + jnp.sum(msk.astype(jnp.int32))

        cnt = lax.fori_loop(0, _NCHUNK, p2_body, jnp.int32(0), unroll=8)
        cbs[pl.ds(cnt, 16)] = jnp.full((16,), inf, jnp.float32)
        cbi[pl.ds(cnt, 16)] = jnp.zeros((16,), jnp.int32)

        def p3_body(g, carry):
            T, Tv = carry
            c0 = g * 16
            sk, si = plsc.sort_key_val(cbs[pl.ds(c0, 16)],
                                       cbi[pl.ds(c0, 16)], descending=True)
            take = sk < T
            lk = jnp.where(take, sk, T)
            lv = jnp.where(take, si, Tv)
            t2, tv2 = plsc.sort_key_val(lk, lv)
            return (t2, tv2)

        t0 = jnp.full((16,), inf, jnp.float32)
        tv0 = jnp.zeros((16,), jnp.int32)
        T, Tv = lax.fori_loop(0, (cnt + 15) // 16, p3_body, (t0, tv0))

        nx = plsc.load_gather(xv, [Tv])
        ny = plsc.load_gather(yv, [Tv])
        nz = plsc.load_gather(zv, [Tv])
        dx = qx - nx
        dy = qy - ny
        dz = qz - nz
        d2 = (dx * dx + dy * dy) + dz * dz

        opx[r] = qx
        opy[r] = qy
        opz[r] = qz
        onx[r] = nx
        ony[r] = ny
        onz[r] = nz
        od2[r] = d2
        oidx[r] = Tv
        return 0

    lax.fori_loop(0, _RPW, row_body, 0)

    pltpu.sync_copy(opx, planes_hbm.at[b, 0, pl.ds(row0, _RPW)])
    pltpu.sync_copy(opy, planes_hbm.at[b, 1, pl.ds(row0, _RPW)])
    pltpu.sync_copy(opz, planes_hbm.at[b, 2, pl.ds(row0, _RPW)])
    pltpu.sync_copy(onx, planes_hbm.at[b, 3, pl.ds(row0, _RPW)])
    pltpu.sync_copy(ony, planes_hbm.at[b, 4, pl.ds(row0, _RPW)])
    pltpu.sync_copy(onz, planes_hbm.at[b, 5, pl.ds(row0, _RPW)])
    pltpu.sync_copy(od2, planes_hbm.at[b, 6, pl.ds(row0, _RPW)])
    pltpu.sync_copy(oidx, idx_hbm.at[b, pl.ds(row0, _RPW)])


@functools.cache
def _get_sc_knn():
    return functools.partial(
        pl.kernel,
        out_type=(jax.ShapeDtypeStruct((_B, 7, _N, _K), jnp.float32),
                  jax.ShapeDtypeStruct((_B, _N, _K), jnp.int32)),
        mesh=plsc.VectorSubcoreMesh(core_axis_name="c", subcore_axis_name="s",
                                    num_cores=_NC, num_subcores=_NS),
        scratch_types=(
            [pltpu.VMEM((_N,), jnp.float32)] * 7
            + [pltpu.VMEM((_N,), jnp.float32),
               pltpu.VMEM((_N + 16,), jnp.float32),
               pltpu.VMEM((_N + 16,), jnp.int32)]
            + [pltpu.VMEM((_RPW, _K), jnp.float32)] * 7
            + [pltpu.VMEM((_RPW, _K), jnp.int32)]
        ),
        compiler_params=pltpu.CompilerParams(use_tc_tiling_on_sc=False,
                                             needs_layout_passes=False),
    )(_sc_knn_body)


def _y1(pl_ref, a1w, b1w, w1d, b1):
    p3 = pl_ref[0, 0:3, :]
    n3 = pl_ref[0, 3:6, :]
    dist = jnp.sqrt(pl_ref[0, 6:7, :])
    y = lax.dot_general(a1w, p3, (((1,), (0,)), ((), ())),
                        precision=lax.Precision.HIGHEST)
    y = y + lax.dot_general(b1w, n3, (((1,), (0,)), ((), ())),
                            precision=lax.Precision.HIGHEST)
    return y + w1d * dist + b1


def _lrelu(x):
    return jnp.where(x >= 0, x, 0.01 * x)


def _accum(s_ref, q_ref, y):
    @pl.when((pl.program_id(0) == 0) & (pl.program_id(1) == 0))
    def _():
        s_ref[...] = jnp.zeros_like(s_ref)
        q_ref[...] = jnp.zeros_like(q_ref)

    s_ref[...] += jnp.sum(y, axis=1, keepdims=True)
    q_ref[...] += jnp.sum(y * y, axis=1, keepdims=True)


def _stats1_body(a1w_ref, b1w_ref, w1d_ref, b1_ref, pl_ref, s_ref, q_ref):
    y1 = _y1(pl_ref, a1w_ref[...], b1w_ref[...], w1d_ref[...], b1_ref[...])
    _accum(s_ref, q_ref, y1)


def _stats2_body(a1w_ref, b1w_ref, w1d_ref, b1_ref, a1_ref, c1_ref,
                 w2_ref, b2_ref, pl_ref, s_ref, q_ref):
    y1 = _y1(pl_ref, a1w_ref[...], b1w_ref[...], w1d_ref[...], b1_ref[...])
    h1 = _lrelu(a1_ref[...] * y1 + c1_ref[...])
    y2 = lax.dot_general(w2_ref[...], h1, (((1,), (0,)), ((), ())),
                         precision=lax.Precision.HIGHEST) + b2_ref[...]
    _accum(s_ref, q_ref, y2)


def _final_body(a1w_ref, b1w_ref, w1d_ref, b1_ref, a1_ref, c1_ref,
                w2_ref, b2_ref, a2_ref, c2_ref, w3_ref, b3_ref,
                pl_ref, out_ref):
    y1 = _y1(pl_ref, a1w_ref[...], b1w_ref[...], w1d_ref[...], b1_ref[...])
    h1 = _lrelu(a1_ref[...] * y1 + c1_ref[...])
    y2 = lax.dot_general(w2_ref[...], h1, (((1,), (0,)), ((), ())),
                         precision=lax.Precision.HIGHEST) + b2_ref[...]
    h2 = _lrelu(a2_ref[...] * y2 + c2_ref[...])
    f = lax.dot_general(w3_ref[...], h2, (((1,), (0,)), ((), ())),
                        precision=lax.Precision.HIGHEST) + b3_ref[...]
    out_ref[0] = f


def _small(shape):
    return pl.BlockSpec(shape, lambda b, e: (0,) * len(shape))


_PLANES_SPEC = pl.BlockSpec((1, 7, _TE), lambda b, e: (b, 0, e))
_GRID = (_B, _NK // _TE)
_STAT_OUT = [jax.ShapeDtypeStruct((64, 1), jnp.float32)] * 2
_STAT_OUT_SPEC = [pl.BlockSpec((64, 1), lambda b, e: (0, 0))] * 2
_W64 = _small((64, 3))
_V64 = _small((64, 1))


@functools.cache
def _get_tc_kernels():
    stats1 = pl.pallas_call(
        _stats1_body,
        grid=_GRID,
        in_specs=[_W64, _W64, _V64, _V64, _PLANES_SPEC],
        out_specs=_STAT_OUT_SPEC,
        out_shape=_STAT_OUT,
    )
    stats2 = pl.pallas_call(
        _stats2_body,
        grid=_GRID,
        in_specs=[_W64, _W64, _V64, _V64, _V64, _V64, _small((64, 64)), _V64,
                  _PLANES_SPEC],
        out_specs=_STAT_OUT_SPEC,
        out_shape=_STAT_OUT,
    )
    final = pl.pallas_call(
        _final_body,
        grid=_GRID,
        in_specs=[_W64, _W64, _V64, _V64, _V64, _V64, _small((64, 64)), _V64,
                  _V64, _V64, _small((64, 64)), _V64, _PLANES_SPEC],
        out_specs=pl.BlockSpec((1, 64, _TE), lambda b, e: (b, 0, e)),
        out_shape=jax.ShapeDtypeStruct((_B, 64, _NK), jnp.float32),
    )
    return stats1, stats2, final


def _bn_fold(s, q, g, be):
    m = s / _M
    v = q / _M - m * m
    a = g / jnp.sqrt(v + 1e-5)
    c = be - m * a
    return a, c


def kernel(xyz, W1, b1, g1, be1, W2, b2, g2, be2, W3, b3):
    xyzT = jnp.transpose(xyz, (0, 2, 1))
    sq = jnp.sum(xyz * xyz, axis=-1)
    xb = xyz.astype(jnp.bfloat16).astype(jnp.float32)
    xbT = jnp.transpose(xb, (0, 2, 1))
    planes, idxp = _get_sc_knn()(xyzT, xbT, sq)
    planes = planes.reshape(_B, 7, _NK)
    _stats1, _stats2, _final = _get_tc_kernels()

    a1w = W1[:, 0:3] + W1[:, 6:9]
    b1w = W1[:, 3:6] - W1[:, 6:9]
    w1d = W1[:, 9:10]
    b1r = b1.reshape(64, 1)
    b2r = b2.reshape(64, 1)
    b3r = b3.reshape(64, 1)

    s1, q1 = _stats1(a1w, b1w, w1d, b1r, planes)
    a1, c1 = _bn_fold(s1, q1, g1.reshape(64, 1), be1.reshape(64, 1))
    s2, q2 = _stats2(a1w, b1w, w1d, b1r, a1, c1, W2, b2r, planes)
    a2, c2 = _bn_fold(s2, q2, g2.reshape(64, 1), be2.reshape(64, 1))
    f = _final(a1w, b1w, w1d, b1r, a1, c1, W2, b2r, a2, c2, W3, b3r, planes)

    return (f.reshape(_B, _DIM_OUT, _N, _K), idxp.reshape(_B, _NK))


# speedup vs baseline: 8.6563x; 1.0701x over previous
import functools

import jax
import jax.numpy as jnp
from jax import lax
from jax.experimental import pallas as pl
from jax.experimental.pallas import tpu as pltpu
from jax.experimental.pallas import tpu_sc as plsc

_B, _N, _C, _K = 4, 4096, 3, 16
_DIM_OUT = 64
_NC, _NS = 2, 16
_NW = _NC * _NS
_RPW = (_B * _N) // _NW
_WPB = _N // _RPW
_NCHUNK = _N // 16
_NK = _N * _K
_TE = 8192
_M = _B * _N * _K


def _sc_knn_body(xyzT_hbm, xbT_hbm, sq_hbm, planes_hbm, idx_hbm,
                 xv, yv, zv, xbv, ybv, zbv, sqv, sbuf, cbs, cbi,
                 opx, opy, opz, onx, ony, onz, od2, oidx):
    cid = lax.axis_index("c")
    sid = lax.axis_index("s")
    wid = sid * _NC + cid
    b = wid // _WPB
    row0 = (wid % _WPB) * _RPW

    pltpu.sync_copy(xyzT_hbm.at[b, 0], xv)
    pltpu.sync_copy(xyzT_hbm.at[b, 1], yv)
    pltpu.sync_copy(xyzT_hbm.at[b, 2], zv)
    pltpu.sync_copy(xbT_hbm.at[b, 0], xbv)
    pltpu.sync_copy(xbT_hbm.at[b, 1], ybv)
    pltpu.sync_copy(xbT_hbm.at[b, 2], zbv)
    pltpu.sync_copy(sq_hbm.at[b], sqv)

    iota16 = lax.iota(jnp.int32, 16)
    inf = jnp.float32(jnp.inf)

    def row_body(r, carry_unused):
        q = row0 + r
        qvec = jnp.full((16,), q, jnp.int32)
        qx = plsc.load_gather(xv, [qvec])
        qy = plsc.load_gather(yv, [qvec])
        qz = plsc.load_gather(zv, [qvec])
        qxb = plsc.load_gather(xbv, [qvec])
        qyb = plsc.load_gather(ybv, [qvec])
        qzb = plsc.load_gather(zbv, [qvec])
        sqi = plsc.load_gather(sqv, [qvec])

        def p1_body(j, m):
            c0 = j * 16
            cx = xbv[pl.ds(c0, 16)]
            cy = ybv[pl.ds(c0, 16)]
            cz = zbv[pl.ds(c0, 16)]
            cs = sqv[pl.ds(c0, 16)]
            dot = ((qxb * cx) + (qyb * cy)) + (qzb * cz)
            score = (sqi + cs) - 2.0 * dot
            sbuf[pl.ds(c0, 16)] = score
            return jnp.minimum(m, score)

        m = lax.fori_loop(0, _NCHUNK, p1_body,
                          jnp.full((16,), inf, jnp.float32), unroll=8)
        tau = jnp.max(m)

        def p2_body(j, off):
            c0 = j * 16
            sc = sbuf[pl.ds(c0, 16)]
            msk = sc <= tau
            plsc.store_compressed(cbs.at[pl.ds(off, 16)], sc, mask=msk)
            plsc.store_compressed(cbi.at[pl.ds(off, 16)], c0 + iota16,
                                  mask=msk)
            return off + plsc.all_reduce_population_count(msk)[0]

        cnt = lax.fori_loop(0, _NCHUNK, p2_body, jnp.int32(0), unroll=8)
        cbs[pl.ds(cnt, 16)] = jnp.full((16,), inf, jnp.float32)
        cbi[pl.ds(cnt, 16)] = jnp.zeros((16,), jnp.int32)

        def p3_body(g, carry):
            T, Tv = carry
            c0 = g * 16
            sk, si = plsc.sort_key_val(cbs[pl.ds(c0, 16)],
                                       cbi[pl.ds(c0, 16)], descending=True)
            take = sk < T
            lk = jnp.where(take, sk, T)
            lv = jnp.where(take, si, Tv)
            t2, tv2 = plsc.sort_key_val(lk, lv)
            return (t2, tv2)

        t0 = jnp.full((16,), inf, jnp.float32)
        tv0 = jnp.zeros((16,), jnp.int32)
        T, Tv = lax.fori_loop(0, (cnt + 15) // 16, p3_body, (t0, tv0))

        nx = plsc.load_gather(xv, [Tv])
        ny = plsc.load_gather(yv, [Tv])
        nz = plsc.load_gather(zv, [Tv])
        dx = qx - nx
        dy = qy - ny
        dz = qz - nz
        d2 = (dx * dx + dy * dy) + dz * dz

        opx[r] = qx
        opy[r] = qy
        opz[r] = qz
        onx[r] = nx
        ony[r] = ny
        onz[r] = nz
        od2[r] = d2
        oidx[r] = Tv
        return 0

    lax.fori_loop(0, _RPW, row_body, 0)

    pltpu.sync_copy(opx, planes_hbm.at[b, 0, pl.ds(row0, _RPW)])
    pltpu.sync_copy(opy, planes_hbm.at[b, 1, pl.ds(row0, _RPW)])
    pltpu.sync_copy(opz, planes_hbm.at[b, 2, pl.ds(row0, _RPW)])
    pltpu.sync_copy(onx, planes_hbm.at[b, 3, pl.ds(row0, _RPW)])
    pltpu.sync_copy(ony, planes_hbm.at[b, 4, pl.ds(row0, _RPW)])
    pltpu.sync_copy(onz, planes_hbm.at[b, 5, pl.ds(row0, _RPW)])
    pltpu.sync_copy(od2, planes_hbm.at[b, 6, pl.ds(row0, _RPW)])
    pltpu.sync_copy(oidx, idx_hbm.at[b, pl.ds(row0, _RPW)])


@functools.cache
def _get_sc_knn():
    return functools.partial(
        pl.kernel,
        out_type=(jax.ShapeDtypeStruct((_B, 7, _N, _K), jnp.float32),
                  jax.ShapeDtypeStruct((_B, _N, _K), jnp.int32)),
        mesh=plsc.VectorSubcoreMesh(core_axis_name="c", subcore_axis_name="s",
                                    num_cores=_NC, num_subcores=_NS),
        scratch_types=(
            [pltpu.VMEM((_N,), jnp.float32)] * 7
            + [pltpu.VMEM((_N,), jnp.float32),
               pltpu.VMEM((_N + 16,), jnp.float32),
               pltpu.VMEM((_N + 16,), jnp.int32)]
            + [pltpu.VMEM((_RPW, _K), jnp.float32)] * 7
            + [pltpu.VMEM((_RPW, _K), jnp.int32)]
        ),
        compiler_params=pltpu.CompilerParams(use_tc_tiling_on_sc=False,
                                             needs_layout_passes=False),
    )(_sc_knn_body)


def _y1(pl_ref, a1w, b1w, w1d, b1):
    p3 = pl_ref[0, 0:3, :]
    n3 = pl_ref[0, 3:6, :]
    dist = jnp.sqrt(pl_ref[0, 6:7, :])
    y = lax.dot_general(a1w, p3, (((1,), (0,)), ((), ())),
                        precision=lax.Precision.HIGHEST)
    y = y + lax.dot_general(b1w, n3, (((1,), (0,)), ((), ())),
                            precision=lax.Precision.HIGHEST)
    return y + w1d * dist + b1


def _lrelu(x):
    return jnp.where(x >= 0, x, 0.01 * x)


def _accum(s_ref, q_ref, y):
    @pl.when((pl.program_id(0) == 0) & (pl.program_id(1) == 0))
    def _():
        s_ref[...] = jnp.zeros_like(s_ref)
        q_ref[...] = jnp.zeros_like(q_ref)

    s_ref[...] += jnp.sum(y, axis=1, keepdims=True)
    q_ref[...] += jnp.sum(y * y, axis=1, keepdims=True)


def _stats1_body(a1w_ref, b1w_ref, w1d_ref, b1_ref, pl_ref, s_ref, q_ref):
    y1 = _y1(pl_ref, a1w_ref[...], b1w_ref[...], w1d_ref[...], b1_ref[...])
    _accum(s_ref, q_ref, y1)


def _stats2_body(a1w_ref, b1w_ref, w1d_ref, b1_ref, a1_ref, c1_ref,
                 w2_ref, b2_ref, pl_ref, s_ref, q_ref):
    y1 = _y1(pl_ref, a1w_ref[...], b1w_ref[...], w1d_ref[...], b1_ref[...])
    h1 = _lrelu(a1_ref[...] * y1 + c1_ref[...])
    y2 = lax.dot_general(w2_ref[...], h1, (((1,), (0,)), ((), ())),
                         precision=lax.Precision.HIGHEST) + b2_ref[...]
    _accum(s_ref, q_ref, y2)


def _final_body(a1w_ref, b1w_ref, w1d_ref, b1_ref, a1_ref, c1_ref,
                w2_ref, b2_ref, a2_ref, c2_ref, w3_ref, b3_ref,
                pl_ref, out_ref):
    y1 = _y1(pl_ref, a1w_ref[...], b1w_ref[...], w1d_ref[...], b1_ref[...])
    h1 = _lrelu(a1_ref[...] * y1 + c1_ref[...])
    y2 = lax.dot_general(w2_ref[...], h1, (((1,), (0,)), ((), ())),
                         precision=lax.Precision.HIGHEST) + b2_ref[...]
    h2 = _lrelu(a2_ref[...] * y2 + c2_ref[...])
    f = lax.dot_general(w3_ref[...], h2, (((1,), (0,)), ((), ())),
                        precision=lax.Precision.HIGHEST) + b3_ref[...]
    out_ref[0] = f


def _small(shape):
    return pl.BlockSpec(shape, lambda b, e: (0,) * len(shape))


_PLANES_SPEC = pl.BlockSpec((1, 7, _TE), lambda b, e: (b, 0, e))
_GRID = (_B, _NK // _TE)
_STAT_OUT = [jax.ShapeDtypeStruct((64, 1), jnp.float32)] * 2
_STAT_OUT_SPEC = [pl.BlockSpec((64, 1), lambda b, e: (0, 0))] * 2
_W64 = _small((64, 3))
_V64 = _small((64, 1))


@functools.cache
def _get_tc_kernels():
    stats1 = pl.pallas_call(
        _stats1_body,
        grid=_GRID,
        in_specs=[_W64, _W64, _V64, _V64, _PLANES_SPEC],
        out_specs=_STAT_OUT_SPEC,
        out_shape=_STAT_OUT,
    )
    stats2 = pl.pallas_call(
        _stats2_body,
        grid=_GRID,
        in_specs=[_W64, _W64, _V64, _V64, _V64, _V64, _small((64, 64)), _V64,
                  _PLANES_SPEC],
        out_specs=_STAT_OUT_SPEC,
        out_shape=_STAT_OUT,
    )
    final = pl.pallas_call(
        _final_body,
        grid=_GRID,
        in_specs=[_W64, _W64, _V64, _V64, _V64, _V64, _small((64, 64)), _V64,
                  _V64, _V64, _small((64, 64)), _V64, _PLANES_SPEC],
        out_specs=pl.BlockSpec((1, 64, _TE), lambda b, e: (b, 0, e)),
        out_shape=jax.ShapeDtypeStruct((_B, 64, _NK), jnp.float32),
    )
    return stats1, stats2, final


def _bn_fold(s, q, g, be):
    m = s / _M
    v = q / _M - m * m
    a = g / jnp.sqrt(v + 1e-5)
    c = be - m * a
    return a, c


def kernel(xyz, W1, b1, g1, be1, W2, b2, g2, be2, W3, b3):
    xyzT = jnp.transpose(xyz, (0, 2, 1))
    sq = jnp.sum(xyz * xyz, axis=-1)
    xb = xyz.astype(jnp.bfloat16).astype(jnp.float32)
    xbT = jnp.transpose(xb, (0, 2, 1))
    planes, idxp = _get_sc_knn()(xyzT, xbT, sq)
    planes = planes.reshape(_B, 7, _NK)
    _stats1, _stats2, _final = _get_tc_kernels()

    a1w = W1[:, 0:3] + W1[:, 6:9]
    b1w = W1[:, 3:6] - W1[:, 6:9]
    w1d = W1[:, 9:10]
    b1r = b1.reshape(64, 1)
    b2r = b2.reshape(64, 1)
    b3r = b3.reshape(64, 1)

    s1, q1 = _stats1(a1w, b1w, w1d, b1r, planes)
    a1, c1 = _bn_fold(s1, q1, g1.reshape(64, 1), be1.reshape(64, 1))
    s2, q2 = _stats2(a1w, b1w, w1d, b1r, a1, c1, W2, b2r, planes)
    a2, c2 = _bn_fold(s2, q2, g2.reshape(64, 1), be2.reshape(64, 1))
    f = _final(a1w, b1w, w1d, b1r, a1, c1, W2, b2r, a2, c2, W3, b3r, planes)

    return (f.reshape(_B, _DIM_OUT, _N, _K), idxp.reshape(_B, _NK))


# speedup vs baseline: 11.7053x; 1.3522x over previous
import functools

import jax
import jax.numpy as jnp
from jax import lax
from jax.experimental import pallas as pl
from jax.experimental.pallas import tpu as pltpu
from jax.experimental.pallas import tpu_sc as plsc

_B, _N, _C, _K = 4, 4096, 3, 16
_DIM_OUT = 64
_NC, _NS = 2, 16
_NW = _NC * _NS
_RPW = (_B * _N) // _NW
_WPB = _N // _RPW
_NCHUNK = _N // 16
_NK = _N * _K
_GR = 4
_TE = 8192
_M = _B * _N * _K


def _sc_knn_body(xyzT_hbm, xbT_hbm, sq_hbm, planes_hbm, idx_hbm,
                 xv, yv, zv, xbv, ybv, zbv, sqv, sb0, sb1, sb2, sb3,
                 cbs, cbi, opx, opy, opz, onx, ony, onz, od2, oidx):
    cid = lax.axis_index("c")
    sid = lax.axis_index("s")
    wid = sid * _NC + cid
    b = wid // _WPB
    row0 = (wid % _WPB) * _RPW

    pltpu.sync_copy(xyzT_hbm.at[b, 0], xv)
    pltpu.sync_copy(xyzT_hbm.at[b, 1], yv)
    pltpu.sync_copy(xyzT_hbm.at[b, 2], zv)
    pltpu.sync_copy(xbT_hbm.at[b, 0], xbv)
    pltpu.sync_copy(xbT_hbm.at[b, 1], ybv)
    pltpu.sync_copy(xbT_hbm.at[b, 2], zbv)
    pltpu.sync_copy(sq_hbm.at[b], sqv)

    iota16 = lax.iota(jnp.int32, 16)
    inf = jnp.float32(jnp.inf)

    sbufs = (sb0, sb1, sb2, sb3)

    def group_body(g, carry_unused):
        r0 = g * _GR
        qsplat = []
        for t in range(_GR):
            qvec = jnp.full((16,), row0 + r0 + t, jnp.int32)
            qsplat.append((
                plsc.load_gather(xv, [qvec]),
                plsc.load_gather(yv, [qvec]),
                plsc.load_gather(zv, [qvec]),
                plsc.load_gather(xbv, [qvec]),
                plsc.load_gather(ybv, [qvec]),
                plsc.load_gather(zbv, [qvec]),
                plsc.load_gather(sqv, [qvec]),
            ))

        def p1_body(j, ms):
            c0 = j * 16
            cx = xbv[pl.ds(c0, 16)]
            cy = ybv[pl.ds(c0, 16)]
            cz = zbv[pl.ds(c0, 16)]
            cs = sqv[pl.ds(c0, 16)]
            out = []
            for t in range(_GR):
                _, _, _, qxb, qyb, qzb, sqi = qsplat[t]
                dot = ((qxb * cx) + (qyb * cy)) + (qzb * cz)
                score = (sqi + cs) - 2.0 * dot
                sbufs[t][pl.ds(c0, 16)] = score
                out.append(jnp.minimum(ms[t], score))
            return tuple(out)

        inf16 = jnp.full((16,), inf, jnp.float32)
        ms = lax.fori_loop(0, _NCHUNK, p1_body, (inf16,) * _GR, unroll=4)

        for t in range(_GR):
            qx, qy, qz = qsplat[t][0], qsplat[t][1], qsplat[t][2]
            sbuf = sbufs[t]
            tau = jnp.max(ms[t])

            def p2_body(j, off):
                c0 = j * 16
                sc = sbuf[pl.ds(c0, 16)]
                msk = sc <= tau
                plsc.store_compressed(cbs.at[pl.ds(off, 16)], sc, mask=msk)
                plsc.store_compressed(cbi.at[pl.ds(off, 16)], c0 + iota16,
                                      mask=msk)
                return off + plsc.all_reduce_population_count(msk)[0]

            cnt = lax.fori_loop(0, _NCHUNK, p2_body, jnp.int32(0), unroll=8)
            cbs[pl.ds(cnt, 16)] = inf16
            cbi[pl.ds(cnt, 16)] = jnp.zeros((16,), jnp.int32)

            def p3_body(u, carry):
                T, Tv = carry
                c0 = u * 16
                sk, si = plsc.sort_key_val(cbs[pl.ds(c0, 16)],
                                           cbi[pl.ds(c0, 16)],
                                           descending=True)
                take = sk < T
                lk = jnp.where(take, sk, T)
                lv = jnp.where(take, si, Tv)
                t2, tv2 = plsc.sort_key_val(lk, lv)
                return (t2, tv2)

            tv0 = jnp.zeros((16,), jnp.int32)
            T, Tv = lax.fori_loop(0, (cnt + 15) // 16, p3_body,
                                  (inf16, tv0))

            nx = plsc.load_gather(xv, [Tv])
            ny = plsc.load_gather(yv, [Tv])
            nz = plsc.load_gather(zv, [Tv])
            dx = qx - nx
            dy = qy - ny
            dz = qz - nz
            d2 = (dx * dx + dy * dy) + dz * dz

            r = r0 + t
            opx[r] = qx
            opy[r] = qy
            opz[r] = qz
            onx[r] = nx
            ony[r] = ny
            onz[r] = nz
            od2[r] = d2
            oidx[r] = Tv
        return 0

    lax.fori_loop(0, _RPW // _GR, group_body, 0)

    pltpu.sync_copy(opx, planes_hbm.at[b, 0, pl.ds(row0, _RPW)])
    pltpu.sync_copy(opy, planes_hbm.at[b, 1, pl.ds(row0, _RPW)])
    pltpu.sync_copy(opz, planes_hbm.at[b, 2, pl.ds(row0, _RPW)])
    pltpu.sync_copy(onx, planes_hbm.at[b, 3, pl.ds(row0, _RPW)])
    pltpu.sync_copy(ony, planes_hbm.at[b, 4, pl.ds(row0, _RPW)])
    pltpu.sync_copy(onz, planes_hbm.at[b, 5, pl.ds(row0, _RPW)])
    pltpu.sync_copy(od2, planes_hbm.at[b, 6, pl.ds(row0, _RPW)])
    pltpu.sync_copy(oidx, idx_hbm.at[b, pl.ds(row0, _RPW)])


@functools.cache
def _get_sc_knn():
    return functools.partial(
        pl.kernel,
        out_type=(jax.ShapeDtypeStruct((_B, 7, _N, _K), jnp.float32),
                  jax.ShapeDtypeStruct((_B, _N, _K), jnp.int32)),
        mesh=plsc.VectorSubcoreMesh(core_axis_name="c", subcore_axis_name="s",
                                    num_cores=_NC, num_subcores=_NS),
        scratch_types=(
            [pltpu.VMEM((_N,), jnp.float32)] * 7
            + [pltpu.VMEM((_N,), jnp.float32)] * _GR
            + [pltpu.VMEM((_N + 16,), jnp.float32),
               pltpu.VMEM((_N + 16,), jnp.int32)]
            + [pltpu.VMEM((_RPW, _K), jnp.float32)] * 7
            + [pltpu.VMEM((_RPW, _K), jnp.int32)]
        ),
        compiler_params=pltpu.CompilerParams(use_tc_tiling_on_sc=False,
                                             needs_layout_passes=False),
    )(_sc_knn_body)


def _y1(pl_ref, a1w, b1w, w1d, b1):
    p3 = pl_ref[0, 0:3, :]
    n3 = pl_ref[0, 3:6, :]
    dist = jnp.sqrt(pl_ref[0, 6:7, :])
    y = lax.dot_general(a1w, p3, (((1,), (0,)), ((), ())),
                        precision=lax.Precision.HIGHEST)
    y = y + lax.dot_general(b1w, n3, (((1,), (0,)), ((), ())),
                            precision=lax.Precision.HIGHEST)
    return y + w1d * dist + b1


def _lrelu(x):
    return jnp.where(x >= 0, x, 0.01 * x)


def _accum(s_ref, q_ref, y):
    @pl.when((pl.program_id(0) == 0) & (pl.program_id(1) == 0))
    def _():
        s_ref[...] = jnp.zeros_like(s_ref)
        q_ref[...] = jnp.zeros_like(q_ref)

    s_ref[...] += jnp.sum(y, axis=1, keepdims=True)
    q_ref[...] += jnp.sum(y * y, axis=1, keepdims=True)


def _stats1_body(a1w_ref, b1w_ref, w1d_ref, b1_ref, pl_ref, s_ref, q_ref):
    y1 = _y1(pl_ref, a1w_ref[...], b1w_ref[...], w1d_ref[...], b1_ref[...])
    _accum(s_ref, q_ref, y1)


def _stats2_body(a1w_ref, b1w_ref, w1d_ref, b1_ref, a1_ref, c1_ref,
                 w2_ref, b2_ref, pl_ref, s_ref, q_ref):
    y1 = _y1(pl_ref, a1w_ref[...], b1w_ref[...], w1d_ref[...], b1_ref[...])
    h1 = _lrelu(a1_ref[...] * y1 + c1_ref[...])
    y2 = lax.dot_general(w2_ref[...], h1, (((1,), (0,)), ((), ())),
                         precision=lax.Precision.HIGHEST) + b2_ref[...]
    _accum(s_ref, q_ref, y2)


def _final_body(a1w_ref, b1w_ref, w1d_ref, b1_ref, a1_ref, c1_ref,
                w2_ref, b2_ref, a2_ref, c2_ref, w3_ref, b3_ref,
                pl_ref, out_ref):
    y1 = _y1(pl_ref, a1w_ref[...], b1w_ref[...], w1d_ref[...], b1_ref[...])
    h1 = _lrelu(a1_ref[...] * y1 + c1_ref[...])
    y2 = lax.dot_general(w2_ref[...], h1, (((1,), (0,)), ((), ())),
                         precision=lax.Precision.HIGHEST) + b2_ref[...]
    h2 = _lrelu(a2_ref[...] * y2 + c2_ref[...])
    f = lax.dot_general(w3_ref[...], h2, (((1,), (0,)), ((), ())),
                        precision=lax.Precision.HIGHEST) + b3_ref[...]
    out_ref[0] = f


def _small(shape):
    return pl.BlockSpec(shape, lambda b, e: (0,) * len(shape))


_PLANES_SPEC = pl.BlockSpec((1, 7, _TE), lambda b, e: (b, 0, e))
_GRID = (_B, _NK // _TE)
_STAT_OUT = [jax.ShapeDtypeStruct((64, 1), jnp.float32)] * 2
_STAT_OUT_SPEC = [pl.BlockSpec((64, 1), lambda b, e: (0, 0))] * 2
_W64 = _small((64, 3))
_V64 = _small((64, 1))


@functools.cache
def _get_tc_kernels():
    stats1 = pl.pallas_call(
        _stats1_body,
        grid=_GRID,
        in_specs=[_W64, _W64, _V64, _V64, _PLANES_SPEC],
        out_specs=_STAT_OUT_SPEC,
        out_shape=_STAT_OUT,
    )
    stats2 = pl.pallas_call(
        _stats2_body,
        grid=_GRID,
        in_specs=[_W64, _W64, _V64, _V64, _V64, _V64, _small((64, 64)), _V64,
                  _PLANES_SPEC],
        out_specs=_STAT_OUT_SPEC,
        out_shape=_STAT_OUT,
    )
    final = pl.pallas_call(
        _final_body,
        grid=_GRID,
        in_specs=[_W64, _W64, _V64, _V64, _V64, _V64, _small((64, 64)), _V64,
                  _V64, _V64, _small((64, 64)), _V64, _PLANES_SPEC],
        out_specs=pl.BlockSpec((1, 64, _TE), lambda b, e: (b, 0, e)),
        out_shape=jax.ShapeDtypeStruct((_B, 64, _NK), jnp.float32),
    )
    return stats1, stats2, final


def _bn_fold(s, q, g, be):
    m = s / _M
    v = q / _M - m * m
    a = g / jnp.sqrt(v + 1e-5)
    c = be - m * a
    return a, c


def kernel(xyz, W1, b1, g1, be1, W2, b2, g2, be2, W3, b3):
    xyzT = jnp.transpose(xyz, (0, 2, 1))
    sq = jnp.sum(xyz * xyz, axis=-1)
    xb = xyz.astype(jnp.bfloat16).astype(jnp.float32)
    xbT = jnp.transpose(xb, (0, 2, 1))
    planes, idxp = _get_sc_knn()(xyzT, xbT, sq)
    planes = planes.reshape(_B, 7, _NK)
    _stats1, _stats2, _final = _get_tc_kernels()

    a1w = W1[:, 0:3] + W1[:, 6:9]
    b1w = W1[:, 3:6] - W1[:, 6:9]
    w1d = W1[:, 9:10]
    b1r = b1.reshape(64, 1)
    b2r = b2.reshape(64, 1)
    b3r = b3.reshape(64, 1)

    s1, q1 = _stats1(a1w, b1w, w1d, b1r, planes)
    a1, c1 = _bn_fold(s1, q1, g1.reshape(64, 1), be1.reshape(64, 1))
    s2, q2 = _stats2(a1w, b1w, w1d, b1r, a1, c1, W2, b2r, planes)
    a2, c2 = _bn_fold(s2, q2, g2.reshape(64, 1), be2.reshape(64, 1))
    f = _final(a1w, b1w, w1d, b1r, a1, c1, W2, b2r, a2, c2, W3, b3r, planes)

    return (f.reshape(_B, _DIM_OUT, _N, _K), idxp.reshape(_B, _NK))


# speedup vs baseline: 12.4399x; 1.0628x over previous
import functools

import jax
import jax.numpy as jnp
from jax import lax
from jax.experimental import pallas as pl
from jax.experimental.pallas import tpu as pltpu
from jax.experimental.pallas import tpu_sc as plsc

_B, _N, _C, _K = 4, 4096, 3, 16
_DIM_OUT = 64
_NC, _NS = 2, 16
_NW = _NC * _NS
_RPW = (_B * _N) // _NW
_WPB = _N // _RPW
_NCHUNK = _N // 16
_NK = _N * _K
_GR = 4
_TE = 16384
_M = _B * _N * _K


def _sc_knn_body(xyzT_hbm, xbT_hbm, sq_hbm, planes_hbm, idx_hbm,
                 xv, yv, zv, xbv, ybv, zbv, sqv, sb0, sb1, sb2, sb3,
                 cbs, cbi, opx, opy, opz, onx, ony, onz, od2, oidx):
    cid = lax.axis_index("c")
    sid = lax.axis_index("s")
    wid = sid * _NC + cid
    b = wid // _WPB
    row0 = (wid % _WPB) * _RPW

    pltpu.sync_copy(xyzT_hbm.at[b, 0], xv)
    pltpu.sync_copy(xyzT_hbm.at[b, 1], yv)
    pltpu.sync_copy(xyzT_hbm.at[b, 2], zv)
    pltpu.sync_copy(xbT_hbm.at[b, 0], xbv)
    pltpu.sync_copy(xbT_hbm.at[b, 1], ybv)
    pltpu.sync_copy(xbT_hbm.at[b, 2], zbv)
    pltpu.sync_copy(sq_hbm.at[b], sqv)

    iota16 = lax.iota(jnp.int32, 16)
    inf = jnp.float32(jnp.inf)

    sbufs = (sb0, sb1, sb2, sb3)

    def group_body(g, carry_unused):
        r0 = g * _GR
        qsplat = []
        for t in range(_GR):
            qvec = jnp.full((16,), row0 + r0 + t, jnp.int32)
            qsplat.append((
                plsc.load_gather(xv, [qvec]),
                plsc.load_gather(yv, [qvec]),
                plsc.load_gather(zv, [qvec]),
                plsc.load_gather(xbv, [qvec]),
                plsc.load_gather(ybv, [qvec]),
                plsc.load_gather(zbv, [qvec]),
                plsc.load_gather(sqv, [qvec]),
            ))

        def p1_body(j, ms):
            c0 = j * 16
            cx = xbv[pl.ds(c0, 16)]
            cy = ybv[pl.ds(c0, 16)]
            cz = zbv[pl.ds(c0, 16)]
            cs = sqv[pl.ds(c0, 16)]
            out = []
            for t in range(_GR):
                _, _, _, qxb, qyb, qzb, sqi = qsplat[t]
                dot = ((qxb * cx) + (qyb * cy)) + (qzb * cz)
                score = (sqi + cs) - 2.0 * dot
                sbufs[t][pl.ds(c0, 16)] = score
                out.append(jnp.minimum(ms[t], score))
            return tuple(out)

        inf16 = jnp.full((16,), inf, jnp.float32)
        ms = lax.fori_loop(0, _NCHUNK, p1_body, (inf16,) * _GR, unroll=4)

        for t in range(_GR):
            qx, qy, qz = qsplat[t][0], qsplat[t][1], qsplat[t][2]
            sbuf = sbufs[t]
            tau = jnp.max(ms[t])

            def p2_body(j, off):
                c0 = j * 16
                sc = sbuf[pl.ds(c0, 16)]
                msk = sc <= tau
                plsc.store_compressed(cbs.at[pl.ds(off, 16)], sc, mask=msk)
                plsc.store_compressed(cbi.at[pl.ds(off, 16)], c0 + iota16,
                                      mask=msk)
                return off + plsc.all_reduce_population_count(msk)[0]

            cnt = lax.fori_loop(0, _NCHUNK, p2_body, jnp.int32(0), unroll=8)
            cbs[pl.ds(cnt, 16)] = inf16
            cbi[pl.ds(cnt, 16)] = jnp.zeros((16,), jnp.int32)

            def p3_body(u, carry):
                T, Tv = carry
                c0 = u * 16
                sk, si = plsc.sort_key_val(cbs[pl.ds(c0, 16)],
                                           cbi[pl.ds(c0, 16)],
                                           descending=True)
                take = sk < T
                lk = jnp.where(take, sk, T)
                lv = jnp.where(take, si, Tv)
                t2, tv2 = plsc.sort_key_val(lk, lv)
                return (t2, tv2)

            tv0 = jnp.zeros((16,), jnp.int32)
            T, Tv = lax.fori_loop(0, (cnt + 15) // 16, p3_body,
                                  (inf16, tv0))

            nx = plsc.load_gather(xv, [Tv])
            ny = plsc.load_gather(yv, [Tv])
            nz = plsc.load_gather(zv, [Tv])
            dx = qx - nx
            dy = qy - ny
            dz = qz - nz
            d2 = (dx * dx + dy * dy) + dz * dz

            r = r0 + t
            opx[r] = qx
            opy[r] = qy
            opz[r] = qz
            onx[r] = nx
            ony[r] = ny
            onz[r] = nz
            od2[r] = d2
            oidx[r] = Tv
        return 0

    lax.fori_loop(0, _RPW // _GR, group_body, 0)

    pltpu.sync_copy(opx, planes_hbm.at[b, 0, pl.ds(row0, _RPW)])
    pltpu.sync_copy(opy, planes_hbm.at[b, 1, pl.ds(row0, _RPW)])
    pltpu.sync_copy(opz, planes_hbm.at[b, 2, pl.ds(row0, _RPW)])
    pltpu.sync_copy(onx, planes_hbm.at[b, 3, pl.ds(row0, _RPW)])
    pltpu.sync_copy(ony, planes_hbm.at[b, 4, pl.ds(row0, _RPW)])
    pltpu.sync_copy(onz, planes_hbm.at[b, 5, pl.ds(row0, _RPW)])
    pltpu.sync_copy(od2, planes_hbm.at[b, 6, pl.ds(row0, _RPW)])
    pltpu.sync_copy(oidx, idx_hbm.at[b, pl.ds(row0, _RPW)])


@functools.cache
def _get_sc_knn():
    return functools.partial(
        pl.kernel,
        out_type=(jax.ShapeDtypeStruct((_B, 7, _N, _K), jnp.float32),
                  jax.ShapeDtypeStruct((_B, _N, _K), jnp.int32)),
        mesh=plsc.VectorSubcoreMesh(core_axis_name="c", subcore_axis_name="s",
                                    num_cores=_NC, num_subcores=_NS),
        scratch_types=(
            [pltpu.VMEM((_N,), jnp.float32)] * 7
            + [pltpu.VMEM((_N,), jnp.float32)] * _GR
            + [pltpu.VMEM((_N + 16,), jnp.float32),
               pltpu.VMEM((_N + 16,), jnp.int32)]
            + [pltpu.VMEM((_RPW, _K), jnp.float32)] * 7
            + [pltpu.VMEM((_RPW, _K), jnp.int32)]
        ),
        compiler_params=pltpu.CompilerParams(use_tc_tiling_on_sc=False,
                                             needs_layout_passes=False),
    )(_sc_knn_body)


def _y1(pl_ref, a1w, b1w, w1d, b1):
    p3 = pl_ref[0, 0:3, :]
    n3 = pl_ref[0, 3:6, :]
    dist = jnp.sqrt(pl_ref[0, 6:7, :])
    y = lax.dot_general(a1w, p3, (((1,), (0,)), ((), ())),
                        precision=lax.Precision.HIGHEST)
    y = y + lax.dot_general(b1w, n3, (((1,), (0,)), ((), ())),
                            precision=lax.Precision.HIGHEST)
    return y + w1d * dist + b1


def _lrelu(x):
    return jnp.where(x >= 0, x, 0.01 * x)


def _accum(s_ref, q_ref, y):
    @pl.when((pl.program_id(0) == 0) & (pl.program_id(1) == 0))
    def _():
        s_ref[...] = jnp.zeros_like(s_ref)
        q_ref[...] = jnp.zeros_like(q_ref)

    s_ref[...] += jnp.sum(y, axis=1, keepdims=True)
    q_ref[...] += jnp.sum(y * y, axis=1, keepdims=True)


def _stats1_body(a1w_ref, b1w_ref, w1d_ref, b1_ref, pl_ref, s_ref, q_ref):
    y1 = _y1(pl_ref, a1w_ref[...], b1w_ref[...], w1d_ref[...], b1_ref[...])
    _accum(s_ref, q_ref, y1)


def _stats2_body(a1w_ref, b1w_ref, w1d_ref, b1_ref, a1_ref, c1_ref,
                 w2_ref, b2_ref, pl_ref, s_ref, q_ref):
    y1 = _y1(pl_ref, a1w_ref[...], b1w_ref[...], w1d_ref[...], b1_ref[...])
    h1 = _lrelu(a1_ref[...] * y1 + c1_ref[...])
    y2 = lax.dot_general(w2_ref[...], h1, (((1,), (0,)), ((), ())),
                         precision=lax.Precision.DEFAULT) + b2_ref[...]
    _accum(s_ref, q_ref, y2)


def _final_body(a1w_ref, b1w_ref, w1d_ref, b1_ref, a1_ref, c1_ref,
                w2_ref, b2_ref, a2_ref, c2_ref, w3_ref, b3_ref,
                pl_ref, out_ref):
    y1 = _y1(pl_ref, a1w_ref[...], b1w_ref[...], w1d_ref[...], b1_ref[...])
    h1 = _lrelu(a1_ref[...] * y1 + c1_ref[...])
    y2 = lax.dot_general(w2_ref[...], h1, (((1,), (0,)), ((), ())),
                         precision=lax.Precision.DEFAULT) + b2_ref[...]
    h2 = _lrelu(a2_ref[...] * y2 + c2_ref[...])
    f = lax.dot_general(w3_ref[...], h2, (((1,), (0,)), ((), ())),
                        precision=lax.Precision.DEFAULT) + b3_ref[...]
    out_ref[0] = f


def _small(shape):
    return pl.BlockSpec(shape, lambda b, e: (0,) * len(shape))


_PLANES_SPEC = pl.BlockSpec((1, 7, _TE), lambda b, e: (b, 0, e))
_GRID = (_B, _NK // _TE)
_STAT_OUT = [jax.ShapeDtypeStruct((64, 1), jnp.float32)] * 2
_STAT_OUT_SPEC = [pl.BlockSpec((64, 1), lambda b, e: (0, 0))] * 2
_W64 = _small((64, 3))
_V64 = _small((64, 1))


@functools.cache
def _get_tc_kernels():
    stats1 = pl.pallas_call(
        _stats1_body,
        grid=_GRID,
        in_specs=[_W64, _W64, _V64, _V64, _PLANES_SPEC],
        out_specs=_STAT_OUT_SPEC,
        out_shape=_STAT_OUT,
    )
    stats2 = pl.pallas_call(
        _stats2_body,
        grid=_GRID,
        in_specs=[_W64, _W64, _V64, _V64, _V64, _V64, _small((64, 64)), _V64,
                  _PLANES_SPEC],
        out_specs=_STAT_OUT_SPEC,
        out_shape=_STAT_OUT,
    )
    final = pl.pallas_call(
        _final_body,
        grid=_GRID,
        in_specs=[_W64, _W64, _V64, _V64, _V64, _V64, _small((64, 64)), _V64,
                  _V64, _V64, _small((64, 64)), _V64, _PLANES_SPEC],
        out_specs=pl.BlockSpec((1, 64, _TE), lambda b, e: (b, 0, e)),
        out_shape=jax.ShapeDtypeStruct((_B, 64, _NK), jnp.float32),
    )
    return stats1, stats2, final


def _bn_fold(s, q, g, be):
    m = s / _M
    v = q / _M - m * m
    a = g / jnp.sqrt(v + 1e-5)
    c = be - m * a
    return a, c


def kernel(xyz, W1, b1, g1, be1, W2, b2, g2, be2, W3, b3):
    xyzT = jnp.transpose(xyz, (0, 2, 1))
    sq = jnp.sum(xyz * xyz, axis=-1)
    xb = xyz.astype(jnp.bfloat16).astype(jnp.float32)
    xbT = jnp.transpose(xb, (0, 2, 1))
    planes, idxp = _get_sc_knn()(xyzT, xbT, sq)
    planes = planes.reshape(_B, 7, _NK)
    _stats1, _stats2, _final = _get_tc_kernels()

    a1w = W1[:, 0:3] + W1[:, 6:9]
    b1w = W1[:, 3:6] - W1[:, 6:9]
    w1d = W1[:, 9:10]
    b1r = b1.reshape(64, 1)
    b2r = b2.reshape(64, 1)
    b3r = b3.reshape(64, 1)

    s1, q1 = _stats1(a1w, b1w, w1d, b1r, planes)
    a1, c1 = _bn_fold(s1, q1, g1.reshape(64, 1), be1.reshape(64, 1))
    s2, q2 = _stats2(a1w, b1w, w1d, b1r, a1, c1, W2, b2r, planes)
    a2, c2 = _bn_fold(s2, q2, g2.reshape(64, 1), be2.reshape(64, 1))
    f = _final(a1w, b1w, w1d, b1r, a1, c1, W2, b2r, a2, c2, W3, b3r, planes)

    return (f.reshape(_B, _DIM_OUT, _N, _K), idxp.reshape(_B, _NK))
